# baseline (speedup 1.0000x reference)
import functools

import jax
import jax.numpy as jnp
from jax.experimental import pallas as pl
from jax.experimental.pallas import tpu as pltpu

_N_S = 10000
_N_T = 10000
_B = 16
_NEG = -3.402823e38


def _lrelu(z, s):
    return jnp.where(z >= 0, z, s * z)


def _linear(x, W, b=None, act=None, bn=1000):
    N, K = x.shape
    F = W.shape[1]
    if b is None:
        b = jnp.zeros((F,), jnp.float32)
    b2 = b.reshape(1, F)

    def body(x_ref, W_ref, b_ref, o_ref):
        z = jnp.dot(x_ref[...], W_ref[...], preferred_element_type=jnp.float32)
        z = z + b_ref[...]
        if act == "lrelu":
            z = _lrelu(z, 0.01)
        elif act == "tanh":
            z = jnp.tanh(z)
        o_ref[...] = z

    return pl.pallas_call(
        body,
        grid=(N // bn,),
        in_specs=[
            pl.BlockSpec((bn, K), lambda i: (i, 0)),
            pl.BlockSpec((K, F), lambda i: (0, 0)),
            pl.BlockSpec((1, F), lambda i: (0, 0)),
        ],
        out_specs=pl.BlockSpec((bn, F), lambda i: (i, 0)),
        out_shape=jax.ShapeDtypeStruct((N, F), jnp.float32),
    )(x, W, b2)


def _pre_call(x, W, att2, bn=1000):
    N, K = x.shape
    F = W.shape[1]

    def body(x_ref, W_ref, att_ref, h_ref, a_ref):
        h = jnp.dot(x_ref[...], W_ref[...], preferred_element_type=jnp.float32)
        h_ref[...] = h
        a_ref[...] = jnp.dot(h, att_ref[...], preferred_element_type=jnp.float32)

    return pl.pallas_call(
        body,
        grid=(N // bn,),
        in_specs=[
            pl.BlockSpec((bn, K), lambda i: (i, 0)),
            pl.BlockSpec((K, F), lambda i: (0, 0)),
            pl.BlockSpec((F, 2), lambda i: (0, 0)),
        ],
        out_specs=[
            pl.BlockSpec((bn, F), lambda i: (i, 0)),
            pl.BlockSpec((bn, 2), lambda i: (i, 0)),
        ],
        out_shape=[
            jax.ShapeDtypeStruct((N, F), jnp.float32),
            jax.ShapeDtypeStruct((N, 2), jnp.float32),
        ],
    )(x, W, att2)


def _attention(x_t, ids, U1_xs, U2W, U2b, W1a, W1b, W1bias, W2W, W2b, bn=1000):
    N, K = x_t.shape

    def body(xt_ref, ids_ref, U1_ref, U2W_ref, U2b_ref, W1a_ref, W1b_ref,
             W1bias_ref, W2W_ref, W2b_ref, ee_ref, ss_ref):
        i = pl.program_id(0)
        oh = (ids_ref[...] == jax.lax.broadcasted_iota(jnp.int32, (1, _B), 1)
              ).astype(jnp.float32)
        u1r = jnp.dot(oh, U1_ref[...], preferred_element_type=jnp.float32)
        u2 = jnp.dot(xt_ref[...], U2W_ref[...],
                     preferred_element_type=jnp.float32) + U2b_ref[...]
        w1v = jnp.tanh(
            jnp.dot(u1r, W1a_ref[...], preferred_element_type=jnp.float32)
            + jnp.dot(u2, W1b_ref[...], preferred_element_type=jnp.float32)
            + W1bias_ref[...])
        ei = jnp.dot(w1v, W2W_ref[...],
                     preferred_element_type=jnp.float32) + W2b_ref[...]
        ee = jnp.exp(ei)
        ee_ref[...] = ee

        @pl.when(i == 0)
        def _():
            ss_ref[...] = jnp.zeros_like(ss_ref)

        ss_ref[...] += jnp.dot(oh.T, ee, preferred_element_type=jnp.float32)

    return pl.pallas_call(
        body,
        grid=(N // bn,),
        in_specs=[
            pl.BlockSpec((bn, K), lambda i: (i, 0)),
            pl.BlockSpec((bn, 1), lambda i: (i, 0)),
            pl.BlockSpec((_B, 150), lambda i: (0, 0)),
            pl.BlockSpec((K, 150), lambda i: (0, 0)),
            pl.BlockSpec((1, 150), lambda i: (0, 0)),
            pl.BlockSpec((150, 150), lambda i: (0, 0)),
            pl.BlockSpec((150, 150), lambda i: (0, 0)),
            pl.BlockSpec((1, 150), lambda i: (0, 0)),
            pl.BlockSpec((150, 1), lambda i: (0, 0)),
            pl.BlockSpec((1, 1), lambda i: (0, 0)),
        ],
        out_specs=[
            pl.BlockSpec((bn, 1), lambda i: (i, 0)),
            pl.BlockSpec((_B, 1), lambda i: (0, 0)),
        ],
        out_shape=[
            jax.ShapeDtypeStruct((N, 1), jnp.float32),
            jax.ShapeDtypeStruct((_B, 1), jnp.float32),
        ],
    )(x_t, ids, U1_xs, U2W, U2b, W1a, W1b, W1bias, W2W, W2b)


def _alpha_norm(ee, ids, ssum, bn=1000):
    N = ee.shape[0]

    def body(ee_ref, ids_ref, ss_ref, a_ref):
        oh = (ids_ref[...] == jax.lax.broadcasted_iota(jnp.int32, (1, _B), 1)
              ).astype(jnp.float32)
        s = jnp.dot(oh, ss_ref[...], preferred_element_type=jnp.float32)
        a_ref[...] = ee_ref[...] / (s + 1e-16)

    return pl.pallas_call(
        body,
        grid=(N // bn,),
        in_specs=[
            pl.BlockSpec((bn, 1), lambda i: (i, 0)),
            pl.BlockSpec((bn, 1), lambda i: (i, 0)),
            pl.BlockSpec((_B, 1), lambda i: (0, 0)),
        ],
        out_specs=pl.BlockSpec((bn, 1), lambda i: (i, 0)),
        out_shape=jax.ShapeDtypeStruct((N, 1), jnp.float32),
    )(ee, ids, ssum)


def _pool_smiles(h, ids, bn=1000):
    N, F = h.shape

    def body(h_ref, ids_ref, ones_ref, mx_ref, sm_ref, ct_ref):
        i = pl.program_id(0)

        @pl.when(i == 0)
        def _():
            mx_ref[...] = jnp.full_like(mx_ref, _NEG)
            sm_ref[...] = jnp.zeros_like(sm_ref)
            ct_ref[...] = jnp.zeros_like(ct_ref)

        oh = (ids_ref[...] == jax.lax.broadcasted_iota(jnp.int32, (1, _B), 1)
              ).astype(jnp.float32)
        hb = h_ref[...]
        sm_ref[...] += jnp.dot(oh.T, hb, preferred_element_type=jnp.float32)
        ct_ref[...] += jnp.dot(oh.T, ones_ref[...],
                               preferred_element_type=jnp.float32)
        for s in range(_B):
            m = oh[:, s:s + 1] > 0
            mx = jnp.max(jnp.where(m, hb, _NEG), axis=0, keepdims=True)
            mx_ref[s:s + 1, :] = jnp.maximum(mx_ref[s:s + 1, :], mx)

    ones = jnp.ones((N, 1), jnp.float32)
    return pl.pallas_call(
        body,
        grid=(N // bn,),
        in_specs=[
            pl.BlockSpec((bn, F), lambda i: (i, 0)),
            pl.BlockSpec((bn, 1), lambda i: (i, 0)),
            pl.BlockSpec((bn, 1), lambda i: (i, 0)),
        ],
        out_specs=[
            pl.BlockSpec((_B, F), lambda i: (0, 0)),
            pl.BlockSpec((_B, F), lambda i: (0, 0)),
            pl.BlockSpec((_B, 1), lambda i: (0, 0)),
        ],
        out_shape=[
            jax.ShapeDtypeStruct((_B, F), jnp.float32),
            jax.ShapeDtypeStruct((_B, F), jnp.float32),
            jax.ShapeDtypeStruct((_B, 1), jnp.float32),
        ],
    )(h, ids, ones)


def _pool_prot(hp, alpha, ids, bn=1000):
    N, F = hp.shape

    def body(h_ref, a_ref, ids_ref, mx_ref):
        i = pl.program_id(0)

        @pl.when(i == 0)
        def _():
            mx_ref[...] = jnp.full_like(mx_ref, _NEG)

        oh = ids_ref[...] == jax.lax.broadcasted_iota(jnp.int32, (1, _B), 1)
        hb = h_ref[...] * a_ref[...]
        for s in range(_B):
            m = oh[:, s:s + 1]
            mx = jnp.max(jnp.where(m, hb, _NEG), axis=0, keepdims=True)
            mx_ref[s:s + 1, :] = jnp.maximum(mx_ref[s:s + 1, :], mx)

    return pl.pallas_call(
        body,
        grid=(N // bn,),
        in_specs=[
            pl.BlockSpec((bn, F), lambda i: (i, 0)),
            pl.BlockSpec((bn, 1), lambda i: (i, 0)),
            pl.BlockSpec((bn, 1), lambda i: (i, 0)),
        ],
        out_specs=pl.BlockSpec((_B, F), lambda i: (0, 0)),
        out_shape=jax.ShapeDtypeStruct((_B, F), jnp.float32),
    )(hp, alpha, ids)


def _smile_out(mx, sm, ct, sW, sb, u1W, u1b):
    def body(mx_ref, sm_ref, ct_ref, sW_ref, sb_ref, u1W_ref, u1b_ref,
             xs_ref, u1_ref):
        gap = sm_ref[...] / jnp.maximum(ct_ref[...], 1.0)
        hs = _lrelu(jnp.concatenate([mx_ref[...], gap], axis=1), 0.01)
        xs = jnp.dot(hs, sW_ref[...],
                     preferred_element_type=jnp.float32) + sb_ref[...]
        xs_ref[...] = xs
        u1_ref[...] = jnp.dot(xs, u1W_ref[...],
                              preferred_element_type=jnp.float32) + u1b_ref[...]

    return pl.pallas_call(
        body,
        out_shape=[
            jax.ShapeDtypeStruct((_B, 500), jnp.float32),
            jax.ShapeDtypeStruct((_B, 150), jnp.float32),
        ],
    )(mx, sm, ct, sW, sb.reshape(1, -1), u1W, u1b.reshape(1, -1))


def _head(mx_p, poW, pob, x_smile, x_esm, W1, b1, W2, b2, W3, b3, W4, b4):
    def body(mx_ref, poW_ref, pob_ref, xs_ref, esm_ref, W1_ref, b1_ref,
             W2_ref, b2_ref, W3_ref, b3_ref, W4_ref, b4_ref, o_ref):
        xp = jnp.dot(_lrelu(mx_ref[...], 0.01), poW_ref[...],
                     preferred_element_type=jnp.float32) + pob_ref[...]
        xc = jnp.concatenate([xp, xs_ref[...], esm_ref[...]], axis=1)
        h = _lrelu(jnp.dot(xc, W1_ref[...],
                           preferred_element_type=jnp.float32) + b1_ref[...], 0.01)
        h = _lrelu(jnp.dot(h, W2_ref[...],
                           preferred_element_type=jnp.float32) + b2_ref[...], 0.01)
        h = _lrelu(jnp.dot(h, W3_ref[...],
                           preferred_element_type=jnp.float32) + b3_ref[...], 0.01)
        o_ref[...] = jnp.dot(h, W4_ref[...],
                             preferred_element_type=jnp.float32) + b4_ref[...]

    return pl.pallas_call(
        body,
        out_shape=jax.ShapeDtypeStruct((_B, 1), jnp.float32),
    )(mx_p, poW, pob.reshape(1, -1), x_smile, x_esm,
      W1, b1.reshape(1, -1), W2, b2.reshape(1, -1),
      W3, b3.reshape(1, -1), W4, b4.reshape(1, -1))


def _gat_msg(a2, ae, src, dst, h, N):
    al = a2[src, 0] + a2[dst, 1]
    if ae is not None:
        al = al + ae
    w = jnp.exp(_lrelu(al, 0.2))
    s = jax.ops.segment_sum(w, dst, num_segments=N)
    num = jax.ops.segment_sum(h[src] * w[:, None], dst, num_segments=N)
    return num / (s[:, None] + 1e-16)


def kernel(x_s, edge_index_s, edge_attr_s, x_s_batch, x_t, edge_index_t,
           x_t_batch, prot_esm, interaction_id, params):
    del interaction_id
    p = params
    src_s, dst_s = edge_index_s[0], edge_index_s[1]
    src_t, dst_t = edge_index_t[0], edge_index_t[1]
    ids_s = x_s_batch.reshape(-1, 1).astype(jnp.int32)
    ids_t = x_t_batch.reshape(-1, 1).astype(jnp.int32)

    Ke = jnp.stack([p[f"s{i}"]["W_e"] @ p[f"s{i}"]["att_e"] for i in range(4)],
                   axis=1)
    ae_all = _linear(edge_attr_s, Ke, bn=1000)

    h = x_s
    for i in range(4):
        pr = p[f"s{i}"]
        att2 = jnp.stack([pr["att_src"], pr["att_dst"]], axis=1)
        hw, a2 = _pre_call(h, pr["W"], att2)
        msg = _gat_msg(a2, ae_all[:, i], src_s, dst_s, hw, _N_S)
        out = msg + h + pr["b"][None, :]
        h = _lrelu(out, 0.01) if i < 3 else out

    mx_s, sm_s, ct_s = _pool_smiles(h, ids_s)
    x_smile, U1_xs = _smile_out(mx_s, sm_s, ct_s, p["s_out"]["W"],
                                p["s_out"]["b"], p["U1"]["W"], p["U1"]["b"])

    W1a = p["W1"]["W"][:150]
    W1b = p["W1"]["W"][150:]
    ee, ssum = _attention(x_t, ids_t, U1_xs, p["U2"]["W"],
                          p["U2"]["b"].reshape(1, -1), W1a, W1b,
                          p["W1"]["b"].reshape(1, -1), p["W2"]["W"],
                          p["W2"]["b"].reshape(1, -1))
    alpha = _alpha_norm(ee, ids_t, ssum)

    hp = x_t
    for i in range(4):
        pr = p[f"p{i}"]
        att2 = jnp.stack([pr["att_src"], pr["att_dst"]], axis=1)
        hw, a2 = _pre_call(hp, pr["W"], att2)
        msg = _gat_msg(a2, None, src_t, dst_t, hw, _N_T)
        if i == 0:
            res = _linear(hp, pr["res_W"])
        else:
            res = hp
        out = msg + res + pr["b"][None, :]
        hp = _lrelu(out, 0.01) if i < 3 else out

    mx_p = _pool_prot(hp, alpha, ids_t)

    out = _head(mx_p, p["p_out"]["W"], p["p_out"]["b"], x_smile, prot_esm,
                p["l1"]["W"], p["l1"]["b"], p["l2"]["W"], p["l2"]["b"],
                p["l3"]["W"], p["l3"]["b"], p["out"]["W"], p["out"]["b"])
    return out, alpha


# speedup vs baseline: 3.0079x; 3.0079x over previous
import functools

import jax
import jax.numpy as jnp
from jax import lax
from jax.experimental import pallas as pl
from jax.experimental.pallas import tpu as pltpu
from jax.experimental.pallas import tpu_sc as plsc

_N = 10000
_B = 16
_NEG = -3.402823e38
_K = 128
_NP = 10240
_HALF = 5120
_AROW = 5760


def _lrelu(z, s):
    return jnp.where(z >= 0, z, s * z)


def _linear(x, W, b=None, act=None, bn=1000):
    N, K = x.shape
    F = W.shape[1]
    if b is None:
        b = jnp.zeros((F,), jnp.float32)
    b2 = b.reshape(1, F)

    def body(x_ref, W_ref, b_ref, o_ref):
        z = jnp.dot(x_ref[...], W_ref[...], preferred_element_type=jnp.float32)
        z = z + b_ref[...]
        if act == "lrelu":
            z = _lrelu(z, 0.01)
        o_ref[...] = z

    return pl.pallas_call(
        body,
        grid=(N // bn,),
        in_specs=[
            pl.BlockSpec((bn, K), lambda i: (i, 0)),
            pl.BlockSpec((K, F), lambda i: (0, 0)),
            pl.BlockSpec((1, F), lambda i: (0, 0)),
        ],
        out_specs=pl.BlockSpec((bn, F), lambda i: (i, 0)),
        out_shape=jax.ShapeDtypeStruct((N, F), jnp.float32),
    )(x, W, b2)


def _pre_call(x, W, att2, bn=1280):
    N, K = x.shape
    F = W.shape[1]

    def body(x_ref, W_ref, att_ref, h_ref, a_ref):
        h = jnp.dot(x_ref[...], W_ref[...], preferred_element_type=jnp.float32)
        h_ref[...] = h
        a_ref[...] = jnp.dot(h, att_ref[...], preferred_element_type=jnp.float32)

    return pl.pallas_call(
        body,
        grid=(N // bn,),
        in_specs=[
            pl.BlockSpec((bn, K), lambda i: (i, 0)),
            pl.BlockSpec((K, F), lambda i: (0, 0)),
            pl.BlockSpec((F, 2), lambda i: (0, 0)),
        ],
        out_specs=[
            pl.BlockSpec((bn, F), lambda i: (i, 0)),
            pl.BlockSpec((bn, 2), lambda i: (i, 0)),
        ],
        out_shape=[
            jax.ShapeDtypeStruct((N, F), jnp.float32),
            jax.ShapeDtypeStruct((N, 2), jnp.float32),
        ],
    )(x, W, att2)


def _attention(x_t, ids, U1_xs, U2W, U2b, W1a, W1b, W1bias, W2W, W2b, bn=1000):
    N, K = x_t.shape

    def body(xt_ref, ids_ref, U1_ref, U2W_ref, U2b_ref, W1a_ref, W1b_ref,
             W1bias_ref, W2W_ref, W2b_ref, ee_ref, ss_ref):
        i = pl.program_id(0)
        oh = (ids_ref[...] == jax.lax.broadcasted_iota(jnp.int32, (1, _B), 1)
              ).astype(jnp.float32)
        u1r = jnp.dot(oh, U1_ref[...], preferred_element_type=jnp.float32)
        u2 = jnp.dot(xt_ref[...], U2W_ref[...],
                     preferred_element_type=jnp.float32) + U2b_ref[...]
        w1v = jnp.tanh(
            jnp.dot(u1r, W1a_ref[...], preferred_element_type=jnp.float32)
            + jnp.dot(u2, W1b_ref[...], preferred_element_type=jnp.float32)
            + W1bias_ref[...])
        ei = jnp.dot(w1v, W2W_ref[...],
                     preferred_element_type=jnp.float32) + W2b_ref[...]
        ee = jnp.exp(ei)
        ee_ref[...] = ee

        @pl.when(i == 0)
        def _():
            ss_ref[...] = jnp.zeros_like(ss_ref)

        ss_ref[...] += jnp.dot(oh.T, ee, preferred_element_type=jnp.float32)

    return pl.pallas_call(
        body,
        grid=(N // bn,),
        in_specs=[
            pl.BlockSpec((bn, K), lambda i: (i, 0)),
            pl.BlockSpec((bn, 1), lambda i: (i, 0)),
            pl.BlockSpec((_B, 150), lambda i: (0, 0)),
            pl.BlockSpec((K, 150), lambda i: (0, 0)),
            pl.BlockSpec((1, 150), lambda i: (0, 0)),
            pl.BlockSpec((150, 150), lambda i: (0, 0)),
            pl.BlockSpec((150, 150), lambda i: (0, 0)),
            pl.BlockSpec((1, 150), lambda i: (0, 0)),
            pl.BlockSpec((150, 1), lambda i: (0, 0)),
            pl.BlockSpec((1, 1), lambda i: (0, 0)),
        ],
        out_specs=[
            pl.BlockSpec((bn, 1), lambda i: (i, 0)),
            pl.BlockSpec((_B, 1), lambda i: (0, 0)),
        ],
        out_shape=[
            jax.ShapeDtypeStruct((N, 1), jnp.float32),
            jax.ShapeDtypeStruct((_B, 1), jnp.float32),
        ],
    )(x_t, ids, U1_xs, U2W, U2b, W1a, W1b, W1bias, W2W, W2b)


def _alpha_norm(ee, ids, ssum, bn=1000):
    N = ee.shape[0]

    def body(ee_ref, ids_ref, ss_ref, a_ref):
        oh = (ids_ref[...] == jax.lax.broadcasted_iota(jnp.int32, (1, _B), 1)
              ).astype(jnp.float32)
        s = jnp.dot(oh, ss_ref[...], preferred_element_type=jnp.float32)
        a_ref[...] = ee_ref[...] / (s + 1e-16)

    return pl.pallas_call(
        body,
        grid=(N // bn,),
        in_specs=[
            pl.BlockSpec((bn, 1), lambda i: (i, 0)),
            pl.BlockSpec((bn, 1), lambda i: (i, 0)),
            pl.BlockSpec((_B, 1), lambda i: (0, 0)),
        ],
        out_specs=pl.BlockSpec((bn, 1), lambda i: (i, 0)),
        out_shape=jax.ShapeDtypeStruct((N, 1), jnp.float32),
    )(ee, ids, ssum)


def _pool_smiles(h, ids, bn=1000):
    F = h.shape[1]

    def body(h_ref, ids_ref, ones_ref, mx_ref, sm_ref, ct_ref):
        i = pl.program_id(0)

        @pl.when(i == 0)
        def _():
            mx_ref[...] = jnp.full_like(mx_ref, _NEG)
            sm_ref[...] = jnp.zeros_like(sm_ref)
            ct_ref[...] = jnp.zeros_like(ct_ref)

        oh = (ids_ref[...] == jax.lax.broadcasted_iota(jnp.int32, (1, _B), 1)
              ).astype(jnp.float32)
        hb = h_ref[...]
        sm_ref[...] += jnp.dot(oh.T, hb, preferred_element_type=jnp.float32)
        ct_ref[...] += jnp.dot(oh.T, ones_ref[...],
                               preferred_element_type=jnp.float32)
        for s in range(_B):
            m = oh[:, s:s + 1] > 0
            mx = jnp.max(jnp.where(m, hb, _NEG), axis=0, keepdims=True)
            mx_ref[s:s + 1, :] = jnp.maximum(mx_ref[s:s + 1, :], mx)

    ones = jnp.ones((_N, 1), jnp.float32)
    return pl.pallas_call(
        body,
        grid=(_N // bn,),
        in_specs=[
            pl.BlockSpec((bn, F), lambda i: (i, 0)),
            pl.BlockSpec((bn, 1), lambda i: (i, 0)),
            pl.BlockSpec((bn, 1), lambda i: (i, 0)),
        ],
        out_specs=[
            pl.BlockSpec((_B, F), lambda i: (0, 0)),
            pl.BlockSpec((_B, F), lambda i: (0, 0)),
            pl.BlockSpec((_B, 1), lambda i: (0, 0)),
        ],
        out_shape=[
            jax.ShapeDtypeStruct((_B, F), jnp.float32),
            jax.ShapeDtypeStruct((_B, F), jnp.float32),
            jax.ShapeDtypeStruct((_B, 1), jnp.float32),
        ],
    )(h, ids, ones)


def _pool_prot(hp, alpha, ids, bn=1000):
    F = hp.shape[1]

    def body(h_ref, a_ref, ids_ref, mx_ref):
        i = pl.program_id(0)

        @pl.when(i == 0)
        def _():
            mx_ref[...] = jnp.full_like(mx_ref, _NEG)

        oh = ids_ref[...] == jax.lax.broadcasted_iota(jnp.int32, (1, _B), 1)
        hb = h_ref[...] * a_ref[...]
        for s in range(_B):
            m = oh[:, s:s + 1]
            mx = jnp.max(jnp.where(m, hb, _NEG), axis=0, keepdims=True)
            mx_ref[s:s + 1, :] = jnp.maximum(mx_ref[s:s + 1, :], mx)

    return pl.pallas_call(
        body,
        grid=(_N // bn,),
        in_specs=[
            pl.BlockSpec((bn, F), lambda i: (i, 0)),
            pl.BlockSpec((bn, 1), lambda i: (i, 0)),
            pl.BlockSpec((bn, 1), lambda i: (i, 0)),
        ],
        out_specs=pl.BlockSpec((_B, F), lambda i: (0, 0)),
        out_shape=jax.ShapeDtypeStruct((_B, F), jnp.float32),
    )(hp, alpha, ids)


def _smile_out(mx, sm, ct, sW, sb, u1W, u1b):
    def body(mx_ref, sm_ref, ct_ref, sW_ref, sb_ref, u1W_ref, u1b_ref,
             xs_ref, u1_ref):
        gap = sm_ref[...] / jnp.maximum(ct_ref[...], 1.0)
        hs = _lrelu(jnp.concatenate([mx_ref[...], gap], axis=1), 0.01)
        xs = jnp.dot(hs, sW_ref[...],
                     preferred_element_type=jnp.float32) + sb_ref[...]
        xs_ref[...] = xs
        u1_ref[...] = jnp.dot(xs, u1W_ref[...],
                              preferred_element_type=jnp.float32) + u1b_ref[...]

    return pl.pallas_call(
        body,
        out_shape=[
            jax.ShapeDtypeStruct((_B, 500), jnp.float32),
            jax.ShapeDtypeStruct((_B, 150), jnp.float32),
        ],
    )(mx, sm, ct, sW, sb.reshape(1, -1), u1W, u1b.reshape(1, -1))


def _head(mx_p, poW, pob, x_smile, x_esm, W1, b1, W2, b2, W3, b3, W4, b4):
    def body(mx_ref, poW_ref, pob_ref, xs_ref, esm_ref, W1_ref, b1_ref,
             W2_ref, b2_ref, W3_ref, b3_ref, W4_ref, b4_ref, o_ref):
        xp = jnp.dot(_lrelu(mx_ref[...], 0.01), poW_ref[...],
                     preferred_element_type=jnp.float32) + pob_ref[...]
        xc = jnp.concatenate([xp, xs_ref[...], esm_ref[...]], axis=1)
        h = _lrelu(jnp.dot(xc, W1_ref[...],
                           preferred_element_type=jnp.float32) + b1_ref[...], 0.01)
        h = _lrelu(jnp.dot(h, W2_ref[...],
                           preferred_element_type=jnp.float32) + b2_ref[...], 0.01)
        h = _lrelu(jnp.dot(h, W3_ref[...],
                           preferred_element_type=jnp.float32) + b3_ref[...], 0.01)
        o_ref[...] = jnp.dot(h, W4_ref[...],
                             preferred_element_type=jnp.float32) + b4_ref[...]

    return pl.pallas_call(
        body,
        out_shape=jax.ShapeDtypeStruct((_B, 1), jnp.float32),
    )(mx_p, poW, pob.reshape(1, -1), x_smile, x_esm,
      W1, b1.reshape(1, -1), W2, b2.reshape(1, -1),
      W3, b3.reshape(1, -1), W4, b4.reshape(1, -1))


def _sc_gat_msg(h, asrc, adst, ae, src, dst, zeros, chunks_per_tile, has_ae):
    mesh = plsc.VectorSubcoreMesh(core_axis_name="c", subcore_axis_name="s")

    @functools.partial(
        pl.kernel, mesh=mesh,
        out_type=[
            jax.ShapeDtypeStruct((2, _AROW, 128), jnp.float32),
            jax.ShapeDtypeStruct((2, 80, 128), jnp.float32),
        ],
        scratch_types=[
            pltpu.VMEM((_NP,), jnp.float32),
            pltpu.VMEM((_NP,), jnp.float32),
            pltpu.VMEM((_K,), jnp.int32),
            pltpu.VMEM((_K,), jnp.int32),
            pltpu.VMEM((_K,), jnp.int32),
            pltpu.VMEM((_K,), jnp.float32),
            pltpu.VMEM((_K,), jnp.float32),
            pltpu.VMEM((_K, 128), jnp.float32),
            pltpu.VMEM((_K, 128), jnp.float32),
            pltpu.VMEM((80, 128), jnp.float32),
            pltpu.VMEM((80,), jnp.int32),
            pltpu.VMEM_SHARED((_AROW, 128), jnp.float32),
            pltpu.VMEM_SHARED((80, 128), jnp.float32),
            pltpu.SemaphoreType.DMA,
        ],
        compiler_params=pltpu.CompilerParams(needs_layout_passes=False),
    )
    def k(h_hbm, asrc_hbm, adst_hbm, ae_hbm, src_hbm, dst_hbm, z_hbm,
          out_hbm, outs_hbm,
          asrc_t, adst_t, srcb, dstb, dstl, aeb, wb, hb, mb, sl, idxb,
          acc, sacc, sem):
        c = lax.axis_index("c")
        t = lax.axis_index("s")
        base_node = c * _HALF
        pltpu.sync_copy(asrc_hbm, asrc_t)
        pltpu.sync_copy(adst_hbm, adst_t)
        pltpu.sync_copy(z_hbm.at[pl.ds(t * 360, 360)],
                        acc.at[pl.ds(t * 360, 360)])
        pltpu.sync_copy(z_hbm.at[pl.ds(0, 80)], sl)

        @pl.when(t == 0)
        def _():
            pltpu.sync_copy(z_hbm.at[pl.ds(0, 80)], sacc)

        for g in range(5):
            idxb[pl.ds(g * 16, 16)] = lax.iota(jnp.int32, 16) + (g * 16)
        plsc.subcore_barrier()

        def chunk_body(ci, _):
            base = (t * chunks_per_tile + ci) * _K
            pltpu.sync_copy(src_hbm.at[pl.ds(base, _K)], srcb)
            pltpu.sync_copy(dst_hbm.at[pl.ds(base, _K)], dstb)
            if has_ae:
                pltpu.sync_copy(ae_hbm.at[pl.ds(base, _K)], aeb)
            pltpu.async_copy(h_hbm.at[srcb], hb, sem).wait()

            def alpha_body(i, _):
                sv = srcb[pl.ds(i * 16, 16)]
                dv = dstb[pl.ds(i * 16, 16)]
                al = (plsc.load_gather(asrc_t, [sv])
                      + plsc.load_gather(adst_t, [dv]))
                if has_ae:
                    al = al + aeb[pl.ds(i * 16, 16)]
                al = 0.6 * al + 0.4 * jnp.abs(al)
                w = jnp.exp(al)
                wb[pl.ds(i * 16, 16)] = w
                plsc.addupdate_scatter(
                    sl, [lax.shift_right_logical(dv, 7),
                         lax.bitwise_and(dv, 127)], w)
                dm = plsc.bitcast(dv - base_node, jnp.uint32)
                d2 = jnp.minimum(dm, jnp.uint32(_HALF))
                dstl[pl.ds(i * 16, 16)] = plsc.bitcast(d2, jnp.int32)
                return 0

            lax.fori_loop(0, _K // 16, alpha_body, 0)

            def row_body(g, e_ids):
                wv = wb[pl.ds(g * 16, 16)]
                for f in range(128):
                    cf = jnp.full((16,), f, jnp.int32)
                    hv = plsc.load_gather(hb, [e_ids, cf])
                    plsc.store_scatter(mb, [e_ids, cf], hv * wv)
                return e_ids + 16

            lax.fori_loop(0, _K // 16, row_body, lax.iota(jnp.int32, 16))

            pltpu.sync_copy(mb, acc.at[dstl], add=True)
            return 0

        lax.fori_loop(0, chunks_per_tile, chunk_body, 0)
        pltpu.sync_copy(sl, sacc.at[idxb], add=True)
        plsc.subcore_barrier()
        pltpu.sync_copy(acc.at[pl.ds(t * 360, 360)],
                        out_hbm.at[c, pl.ds(t * 360, 360)])

        @pl.when(t == 0)
        def _():
            pltpu.sync_copy(sacc, outs_hbm.at[c])

    return k(h, asrc, adst, ae, src, dst, zeros)


def _msg_parts(h, a2, ae, src, dst, zeros, cpt, has_ae):
    parts, sparts = _sc_gat_msg(h, a2[:, 0] + 0.0, a2[:, 1] + 0.0,
                                ae, src, dst, zeros, cpt, has_ae)
    return parts, sparts.reshape(2, _NP, 1)


_BC = 320


def _layer_step(parts, s2, res, b, W, att2, act):
    F = W.shape[1]

    def body(p_ref, s_ref, res_ref, b_ref, W_ref, att_ref,
             x_ref, h_ref, a_ref):
        num = p_ref[0]
        s = s_ref[0]
        x = num / (s + 1e-16) + res_ref[...] + b_ref[...]
        if act == "lrelu":
            x = _lrelu(x, 0.01)
        x_ref[...] = x
        h = jnp.dot(x, W_ref[...], preferred_element_type=jnp.float32)
        h_ref[...] = h
        a_ref[...] = jnp.dot(h, att_ref[...], preferred_element_type=jnp.float32)

    return pl.pallas_call(
        body,
        grid=(_NP // _BC,),
        in_specs=[
            pl.BlockSpec((1, _BC, 128), lambda j: (j // 16, j % 16, 0)),
            pl.BlockSpec((1, _BC, 1), lambda j: (j // 16, j, 0)),
            pl.BlockSpec((_BC, 128), lambda j: (j, 0)),
            pl.BlockSpec((1, 128), lambda j: (0, 0)),
            pl.BlockSpec((128, F), lambda j: (0, 0)),
            pl.BlockSpec((F, 2), lambda j: (0, 0)),
        ],
        out_specs=[
            pl.BlockSpec((_BC, 128), lambda j: (j, 0)),
            pl.BlockSpec((_BC, F), lambda j: (j, 0)),
            pl.BlockSpec((_BC, 2), lambda j: (j, 0)),
        ],
        out_shape=[
            jax.ShapeDtypeStruct((_NP, 128), jnp.float32),
            jax.ShapeDtypeStruct((_NP, F), jnp.float32),
            jax.ShapeDtypeStruct((_NP, 2), jnp.float32),
        ],
    )(parts, s2, res, b.reshape(1, -1), W, att2)


def _combine(parts, s2, res, b):
    def body(p_ref, s_ref, res_ref, b_ref, x_ref):
        num = p_ref[0]
        s = s_ref[0]
        x_ref[...] = num / (s + 1e-16) + res_ref[...] + b_ref[...]

    return pl.pallas_call(
        body,
        grid=(_NP // _BC,),
        in_specs=[
            pl.BlockSpec((1, _BC, 128), lambda j: (j // 16, j % 16, 0)),
            pl.BlockSpec((1, _BC, 1), lambda j: (j // 16, j, 0)),
            pl.BlockSpec((_BC, 128), lambda j: (j, 0)),
            pl.BlockSpec((1, 128), lambda j: (0, 0)),
        ],
        out_specs=pl.BlockSpec((_BC, 128), lambda j: (j, 0)),
        out_shape=jax.ShapeDtypeStruct((_NP, 128), jnp.float32),
    )(parts, s2, res, b.reshape(1, -1))


def _pad_edges(src, dst, e_pad):
    pad = e_pad - src.shape[0]
    src_p = jnp.concatenate([src, jnp.zeros((pad,), src.dtype)])
    dst_p = jnp.concatenate([dst, jnp.full((pad,), _N, dst.dtype)])
    return src_p, dst_p


def _pad_rows(x):
    return jnp.concatenate(
        [x, jnp.zeros((_NP - x.shape[0], x.shape[1]), x.dtype)])


def kernel(x_s, edge_index_s, edge_attr_s, x_s_batch, x_t, edge_index_t,
           x_t_batch, prot_esm, interaction_id, params):
    del interaction_id
    p = params
    ids_s = x_s_batch.reshape(-1, 1).astype(jnp.int32)
    ids_t = x_t_batch.reshape(-1, 1).astype(jnp.int32)

    cpt_s = 79
    cpt_t = 157
    src_s, dst_s = _pad_edges(edge_index_s[0], edge_index_s[1], 16 * cpt_s * _K)
    src_t, dst_t = _pad_edges(edge_index_t[0], edge_index_t[1], 16 * cpt_t * _K)
    zeros = jnp.zeros((_NP, 128), jnp.float32)

    Ke = jnp.stack([p[f"s{i}"]["W_e"] @ p[f"s{i}"]["att_e"] for i in range(4)],
                   axis=1)
    ae_all = _linear(edge_attr_s, Ke, bn=1000)
    ae_pad = jnp.zeros((16 * cpt_s * _K, 4), jnp.float32)
    ae_pad = ae_pad.at[:ae_all.shape[0]].set(ae_all)

    x = _pad_rows(x_s)
    pr = p["s0"]
    att2 = jnp.stack([pr["att_src"], pr["att_dst"]], axis=1)
    h, a2 = _pre_call(x, pr["W"], att2)
    for i in range(4):
        pr = p[f"s{i}"]
        parts, s2 = _msg_parts(h, a2, ae_pad[:, i], src_s, dst_s,
                               zeros, cpt_s, True)
        if i < 3:
            nxt = p[f"s{i + 1}"]
            att2 = jnp.stack([nxt["att_src"], nxt["att_dst"]], axis=1)
            x, h, a2 = _layer_step(parts, s2, x, pr["b"], nxt["W"], att2,
                                   "lrelu")
        else:
            h = _combine(parts, s2, x, pr["b"])

    mx_s, sm_s, ct_s = _pool_smiles(h[:_N], ids_s)
    x_smile, U1_xs = _smile_out(mx_s, sm_s, ct_s, p["s_out"]["W"],
                                p["s_out"]["b"], p["U1"]["W"], p["U1"]["b"])

    W1a = p["W1"]["W"][:150]
    W1b = p["W1"]["W"][150:]
    ee, ssum = _attention(x_t, ids_t, U1_xs, p["U2"]["W"],
                          p["U2"]["b"].reshape(1, -1), W1a, W1b,
                          p["W1"]["b"].reshape(1, -1), p["W2"]["W"],
                          p["W2"]["b"].reshape(1, -1))
    alpha = _alpha_norm(ee, ids_t, ssum)

    xt_pad = _pad_rows(x_t)
    pr = p["p0"]
    att2 = jnp.stack([pr["att_src"], pr["att_dst"]], axis=1)
    hp, a2 = _pre_call(xt_pad, pr["W"], att2)
    xp = _linear(xt_pad, pr["res_W"], bn=1280)
    for i in range(4):
        pr = p[f"p{i}"]
        parts, s2 = _msg_parts(hp, a2, src_t, src_t, dst_t,
                               zeros, cpt_t, False)
        if i < 3:
            nxt = p[f"p{i + 1}"]
            att2 = jnp.stack([nxt["att_src"], nxt["att_dst"]], axis=1)
            xp, hp, a2 = _layer_step(parts, s2, xp, pr["b"], nxt["W"], att2,
                                     "lrelu")
        else:
            hp = _combine(parts, s2, xp, pr["b"])

    mx_p = _pool_prot(hp[:_N], alpha, ids_t)

    out = _head(mx_p, p["p_out"]["W"], p["p_out"]["b"], x_smile, prot_esm,
                p["l1"]["W"], p["l1"]["b"], p["l2"]["W"], p["l2"]["b"],
                p["l3"]["W"], p["l3"]["b"], p["out"]["W"], p["out"]["b"])
    return out, alpha


# speedup vs baseline: 3.2136x; 1.0684x over previous
import functools

import jax
import jax.numpy as jnp
from jax import lax
from jax.experimental import pallas as pl
from jax.experimental.pallas import tpu as pltpu
from jax.experimental.pallas import tpu_sc as plsc

_N = 10000
_B = 16
_NEG = -3.402823e38
_K = 128
_NP = 10240
_HALF = 5120
_AROW = 5760


def _lrelu(z, s):
    return jnp.where(z >= 0, z, s * z)


def _linear(x, W, b=None, act=None, bn=1000):
    N, K = x.shape
    F = W.shape[1]
    if b is None:
        b = jnp.zeros((F,), jnp.float32)
    b2 = b.reshape(1, F)

    def body(x_ref, W_ref, b_ref, o_ref):
        z = jnp.dot(x_ref[...], W_ref[...], preferred_element_type=jnp.float32)
        z = z + b_ref[...]
        if act == "lrelu":
            z = _lrelu(z, 0.01)
        o_ref[...] = z

    return pl.pallas_call(
        body,
        grid=(N // bn,),
        in_specs=[
            pl.BlockSpec((bn, K), lambda i: (i, 0)),
            pl.BlockSpec((K, F), lambda i: (0, 0)),
            pl.BlockSpec((1, F), lambda i: (0, 0)),
        ],
        out_specs=pl.BlockSpec((bn, F), lambda i: (i, 0)),
        out_shape=jax.ShapeDtypeStruct((N, F), jnp.float32),
    )(x, W, b2)


def _pre_call(x, W, att2, bn=1280):
    N, K = x.shape
    F = W.shape[1]

    def body(x_ref, W_ref, att_ref, h_ref, a_ref):
        h = jnp.dot(x_ref[...], W_ref[...], preferred_element_type=jnp.float32)
        h_ref[...] = h
        a_ref[...] = jnp.dot(h, att_ref[...], preferred_element_type=jnp.float32)

    return pl.pallas_call(
        body,
        grid=(N // bn,),
        in_specs=[
            pl.BlockSpec((bn, K), lambda i: (i, 0)),
            pl.BlockSpec((K, F), lambda i: (0, 0)),
            pl.BlockSpec((F, 2), lambda i: (0, 0)),
        ],
        out_specs=[
            pl.BlockSpec((bn, F), lambda i: (i, 0)),
            pl.BlockSpec((bn, 2), lambda i: (i, 0)),
        ],
        out_shape=[
            jax.ShapeDtypeStruct((N, F), jnp.float32),
            jax.ShapeDtypeStruct((N, 2), jnp.float32),
        ],
    )(x, W, att2)


def _attention(x_t, ids, U1_xs, U2W, U2b, W1a, W1b, W1bias, W2W, W2b, bn=1000):
    N, K = x_t.shape

    def body(xt_ref, ids_ref, U1_ref, U2W_ref, U2b_ref, W1a_ref, W1b_ref,
             W1bias_ref, W2W_ref, W2b_ref, ee_ref, ss_ref):
        i = pl.program_id(0)
        oh = (ids_ref[...] == jax.lax.broadcasted_iota(jnp.int32, (1, _B), 1)
              ).astype(jnp.float32)
        u1r = jnp.dot(oh, U1_ref[...], preferred_element_type=jnp.float32)
        u2 = jnp.dot(xt_ref[...], U2W_ref[...],
                     preferred_element_type=jnp.float32) + U2b_ref[...]
        w1v = jnp.tanh(
            jnp.dot(u1r, W1a_ref[...], preferred_element_type=jnp.float32)
            + jnp.dot(u2, W1b_ref[...], preferred_element_type=jnp.float32)
            + W1bias_ref[...])
        ei = jnp.dot(w1v, W2W_ref[...],
                     preferred_element_type=jnp.float32) + W2b_ref[...]
        ee = jnp.exp(ei)
        ee_ref[...] = ee

        @pl.when(i == 0)
        def _():
            ss_ref[...] = jnp.zeros_like(ss_ref)

        ss_ref[...] += jnp.dot(oh.T, ee, preferred_element_type=jnp.float32)

    return pl.pallas_call(
        body,
        grid=(N // bn,),
        in_specs=[
            pl.BlockSpec((bn, K), lambda i: (i, 0)),
            pl.BlockSpec((bn, 1), lambda i: (i, 0)),
            pl.BlockSpec((_B, 150), lambda i: (0, 0)),
            pl.BlockSpec((K, 150), lambda i: (0, 0)),
            pl.BlockSpec((1, 150), lambda i: (0, 0)),
            pl.BlockSpec((150, 150), lambda i: (0, 0)),
            pl.BlockSpec((150, 150), lambda i: (0, 0)),
            pl.BlockSpec((1, 150), lambda i: (0, 0)),
            pl.BlockSpec((150, 1), lambda i: (0, 0)),
            pl.BlockSpec((1, 1), lambda i: (0, 0)),
        ],
        out_specs=[
            pl.BlockSpec((bn, 1), lambda i: (i, 0)),
            pl.BlockSpec((_B, 1), lambda i: (0, 0)),
        ],
        out_shape=[
            jax.ShapeDtypeStruct((N, 1), jnp.float32),
            jax.ShapeDtypeStruct((_B, 1), jnp.float32),
        ],
    )(x_t, ids, U1_xs, U2W, U2b, W1a, W1b, W1bias, W2W, W2b)


def _alpha_norm(ee, ids, ssum, bn=1000):
    N = ee.shape[0]

    def body(ee_ref, ids_ref, ss_ref, a_ref):
        oh = (ids_ref[...] == jax.lax.broadcasted_iota(jnp.int32, (1, _B), 1)
              ).astype(jnp.float32)
        s = jnp.dot(oh, ss_ref[...], preferred_element_type=jnp.float32)
        a_ref[...] = ee_ref[...] / (s + 1e-16)

    return pl.pallas_call(
        body,
        grid=(N // bn,),
        in_specs=[
            pl.BlockSpec((bn, 1), lambda i: (i, 0)),
            pl.BlockSpec((bn, 1), lambda i: (i, 0)),
            pl.BlockSpec((_B, 1), lambda i: (0, 0)),
        ],
        out_specs=pl.BlockSpec((bn, 1), lambda i: (i, 0)),
        out_shape=jax.ShapeDtypeStruct((N, 1), jnp.float32),
    )(ee, ids, ssum)


def _pool_smiles(h, ids, bn=1000):
    F = h.shape[1]

    def body(h_ref, ids_ref, ones_ref, mx_ref, sm_ref, ct_ref):
        i = pl.program_id(0)

        @pl.when(i == 0)
        def _():
            mx_ref[...] = jnp.full_like(mx_ref, _NEG)
            sm_ref[...] = jnp.zeros_like(sm_ref)
            ct_ref[...] = jnp.zeros_like(ct_ref)

        oh = (ids_ref[...] == jax.lax.broadcasted_iota(jnp.int32, (1, _B), 1)
              ).astype(jnp.float32)
        hb = h_ref[...]
        sm_ref[...] += jnp.dot(oh.T, hb, preferred_element_type=jnp.float32)
        ct_ref[...] += jnp.dot(oh.T, ones_ref[...],
                               preferred_element_type=jnp.float32)
        for s in range(_B):
            m = oh[:, s:s + 1] > 0
            mx = jnp.max(jnp.where(m, hb, _NEG), axis=0, keepdims=True)
            mx_ref[s:s + 1, :] = jnp.maximum(mx_ref[s:s + 1, :], mx)

    ones = jnp.ones((_N, 1), jnp.float32)
    return pl.pallas_call(
        body,
        grid=(_N // bn,),
        in_specs=[
            pl.BlockSpec((bn, F), lambda i: (i, 0)),
            pl.BlockSpec((bn, 1), lambda i: (i, 0)),
            pl.BlockSpec((bn, 1), lambda i: (i, 0)),
        ],
        out_specs=[
            pl.BlockSpec((_B, F), lambda i: (0, 0)),
            pl.BlockSpec((_B, F), lambda i: (0, 0)),
            pl.BlockSpec((_B, 1), lambda i: (0, 0)),
        ],
        out_shape=[
            jax.ShapeDtypeStruct((_B, F), jnp.float32),
            jax.ShapeDtypeStruct((_B, F), jnp.float32),
            jax.ShapeDtypeStruct((_B, 1), jnp.float32),
        ],
    )(h, ids, ones)


def _pool_prot(hp, alpha, ids, bn=1000):
    F = hp.shape[1]

    def body(h_ref, a_ref, ids_ref, mx_ref):
        i = pl.program_id(0)

        @pl.when(i == 0)
        def _():
            mx_ref[...] = jnp.full_like(mx_ref, _NEG)

        oh = ids_ref[...] == jax.lax.broadcasted_iota(jnp.int32, (1, _B), 1)
        hb = h_ref[...] * a_ref[...]
        for s in range(_B):
            m = oh[:, s:s + 1]
            mx = jnp.max(jnp.where(m, hb, _NEG), axis=0, keepdims=True)
            mx_ref[s:s + 1, :] = jnp.maximum(mx_ref[s:s + 1, :], mx)

    return pl.pallas_call(
        body,
        grid=(_N // bn,),
        in_specs=[
            pl.BlockSpec((bn, F), lambda i: (i, 0)),
            pl.BlockSpec((bn, 1), lambda i: (i, 0)),
            pl.BlockSpec((bn, 1), lambda i: (i, 0)),
        ],
        out_specs=pl.BlockSpec((_B, F), lambda i: (0, 0)),
        out_shape=jax.ShapeDtypeStruct((_B, F), jnp.float32),
    )(hp, alpha, ids)


def _smile_out(mx, sm, ct, sW, sb, u1W, u1b):
    def body(mx_ref, sm_ref, ct_ref, sW_ref, sb_ref, u1W_ref, u1b_ref,
             xs_ref, u1_ref):
        gap = sm_ref[...] / jnp.maximum(ct_ref[...], 1.0)
        hs = _lrelu(jnp.concatenate([mx_ref[...], gap], axis=1), 0.01)
        xs = jnp.dot(hs, sW_ref[...],
                     preferred_element_type=jnp.float32) + sb_ref[...]
        xs_ref[...] = xs
        u1_ref[...] = jnp.dot(xs, u1W_ref[...],
                              preferred_element_type=jnp.float32) + u1b_ref[...]

    return pl.pallas_call(
        body,
        out_shape=[
            jax.ShapeDtypeStruct((_B, 500), jnp.float32),
            jax.ShapeDtypeStruct((_B, 150), jnp.float32),
        ],
    )(mx, sm, ct, sW, sb.reshape(1, -1), u1W, u1b.reshape(1, -1))


def _head(mx_p, poW, pob, x_smile, x_esm, W1, b1, W2, b2, W3, b3, W4, b4):
    def body(mx_ref, poW_ref, pob_ref, xs_ref, esm_ref, W1_ref, b1_ref,
             W2_ref, b2_ref, W3_ref, b3_ref, W4_ref, b4_ref, o_ref):
        xp = jnp.dot(_lrelu(mx_ref[...], 0.01), poW_ref[...],
                     preferred_element_type=jnp.float32) + pob_ref[...]
        xc = jnp.concatenate([xp, xs_ref[...], esm_ref[...]], axis=1)
        h = _lrelu(jnp.dot(xc, W1_ref[...],
                           preferred_element_type=jnp.float32) + b1_ref[...], 0.01)
        h = _lrelu(jnp.dot(h, W2_ref[...],
                           preferred_element_type=jnp.float32) + b2_ref[...], 0.01)
        h = _lrelu(jnp.dot(h, W3_ref[...],
                           preferred_element_type=jnp.float32) + b3_ref[...], 0.01)
        o_ref[...] = jnp.dot(h, W4_ref[...],
                             preferred_element_type=jnp.float32) + b4_ref[...]

    return pl.pallas_call(
        body,
        out_shape=jax.ShapeDtypeStruct((_B, 1), jnp.float32),
    )(mx_p, poW, pob.reshape(1, -1), x_smile, x_esm,
      W1, b1.reshape(1, -1), W2, b2.reshape(1, -1),
      W3, b3.reshape(1, -1), W4, b4.reshape(1, -1))


def _sc_gat_msg(h, asrc, adst, ae, src, dst, zeros, chunks_per_tile, has_ae):
    mesh = plsc.VectorSubcoreMesh(core_axis_name="c", subcore_axis_name="s")

    @functools.partial(
        pl.kernel, mesh=mesh,
        out_type=[
            jax.ShapeDtypeStruct((2, _AROW, 128), jnp.float32),
            jax.ShapeDtypeStruct((2, 80, 128), jnp.float32),
        ],
        scratch_types=[
            pltpu.VMEM((_NP,), jnp.float32),
            pltpu.VMEM((_NP,), jnp.float32),
            pltpu.VMEM((_K,), jnp.int32),
            pltpu.VMEM((_K,), jnp.int32),
            pltpu.VMEM((_K,), jnp.float32),
            pltpu.VMEM((_K, 128), jnp.float32),
            pltpu.VMEM((_K,), jnp.int32),
            pltpu.VMEM((_K,), jnp.int32),
            pltpu.VMEM((_K,), jnp.float32),
            pltpu.VMEM((_K, 128), jnp.float32),
            pltpu.VMEM((_K,), jnp.int32),
            pltpu.VMEM((_K,), jnp.float32),
            pltpu.VMEM((_K, 128), jnp.float32),
            pltpu.VMEM((80, 128), jnp.float32),
            pltpu.VMEM((80,), jnp.int32),
            pltpu.VMEM_SHARED((_AROW, 128), jnp.float32),
            pltpu.VMEM_SHARED((80, 128), jnp.float32),
            pltpu.SemaphoreType.DMA,
            pltpu.SemaphoreType.DMA,
        ],
        compiler_params=pltpu.CompilerParams(needs_layout_passes=False),
    )
    def k(h_hbm, asrc_hbm, adst_hbm, ae_hbm, src_hbm, dst_hbm, z_hbm,
          out_hbm, outs_hbm,
          asrc_t, adst_t, srcb0, dstb0, aeb0, hb0, srcb1, dstb1, aeb1, hb1,
          dstl, wb, mb, sl, idxb, acc, sacc, sem0, sem1):
        c = lax.axis_index("c")
        t = lax.axis_index("s")
        base_node = c * _HALF
        pltpu.sync_copy(asrc_hbm, asrc_t)
        pltpu.sync_copy(adst_hbm, adst_t)
        pltpu.sync_copy(z_hbm.at[pl.ds(t * 360, 360)],
                        acc.at[pl.ds(t * 360, 360)])
        pltpu.sync_copy(z_hbm.at[pl.ds(0, 80)], sl)

        @pl.when(t == 0)
        def _():
            pltpu.sync_copy(z_hbm.at[pl.ds(0, 80)], sacc)

        for g in range(5):
            idxb[pl.ds(g * 16, 16)] = lax.iota(jnp.int32, 16) + (g * 16)
        plsc.subcore_barrier()

        def prefetch(ck, srcb, dstb, aeb, hb, sem):
            base = (t * chunks_per_tile + ck) * _K
            pltpu.sync_copy(src_hbm.at[pl.ds(base, _K)], srcb)
            pltpu.sync_copy(dst_hbm.at[pl.ds(base, _K)], dstb)
            if has_ae:
                pltpu.sync_copy(ae_hbm.at[pl.ds(base, _K)], aeb)
            pltpu.async_copy(h_hbm.at[srcb], hb, sem)

        def do_chunk(srcb, dstb, aeb, hb, sem):
            pltpu.make_async_copy(h_hbm.at[srcb], hb, sem).wait()

            def alpha_body(i, _):
                sv = srcb[pl.ds(i * 16, 16)]
                dv = dstb[pl.ds(i * 16, 16)]
                al = (plsc.load_gather(asrc_t, [sv])
                      + plsc.load_gather(adst_t, [dv]))
                if has_ae:
                    al = al + aeb[pl.ds(i * 16, 16)]
                al = 0.6 * al + 0.4 * jnp.abs(al)
                w = jnp.exp(al)
                wb[pl.ds(i * 16, 16)] = w
                plsc.addupdate_scatter(
                    sl, [lax.shift_right_logical(dv, 7),
                         lax.bitwise_and(dv, 127)], w)
                dm = plsc.bitcast(dv - base_node, jnp.uint32)
                d2 = jnp.minimum(dm, jnp.uint32(_HALF))
                dstl[pl.ds(i * 16, 16)] = plsc.bitcast(d2, jnp.int32)
                return 0

            lax.fori_loop(0, _K // 16, alpha_body, 0)

            def row_body(g, e_ids):
                wv = wb[pl.ds(g * 16, 16)]
                for f in range(128):
                    cf = jnp.full((16,), f, jnp.int32)
                    hv = plsc.load_gather(hb, [e_ids, cf])
                    plsc.store_scatter(mb, [e_ids, cf], hv * wv)
                return e_ids + 16

            lax.fori_loop(0, _K // 16, row_body, lax.iota(jnp.int32, 16))

            pltpu.sync_copy(mb, acc.at[dstl], add=True)

        half = chunks_per_tile // 2
        prefetch(0, srcb0, dstb0, aeb0, hb0, sem0)

        def pair_body(j, _):
            a = 2 * j
            prefetch(a + 1, srcb1, dstb1, aeb1, hb1, sem1)
            do_chunk(srcb0, dstb0, aeb0, hb0, sem0)

            @pl.when(j + 1 < half)
            def _():
                prefetch(a + 2, srcb0, dstb0, aeb0, hb0, sem0)

            do_chunk(srcb1, dstb1, aeb1, hb1, sem1)
            return 0

        lax.fori_loop(0, half, pair_body, 0)
        pltpu.sync_copy(sl, sacc.at[idxb], add=True)
        plsc.subcore_barrier()
        pltpu.sync_copy(acc.at[pl.ds(t * 360, 360)],
                        out_hbm.at[c, pl.ds(t * 360, 360)])

        @pl.when(t == 0)
        def _():
            pltpu.sync_copy(sacc, outs_hbm.at[c])

    return k(h, asrc, adst, ae, src, dst, zeros)


def _msg_parts(h, a2, ae, src, dst, zeros, cpt, has_ae):
    parts, sparts = _sc_gat_msg(h, a2[:, 0] + 0.0, a2[:, 1] + 0.0,
                                ae, src, dst, zeros, cpt, has_ae)
    return parts, sparts.reshape(2, _NP, 1)


_BC = 320


def _layer_step(parts, s2, res, b, W, att2, act):
    F = W.shape[1]

    def body(p_ref, s_ref, res_ref, b_ref, W_ref, att_ref,
             x_ref, h_ref, a_ref):
        num = p_ref[0]
        s = s_ref[0]
        x = num / (s + 1e-16) + res_ref[...] + b_ref[...]
        if act == "lrelu":
            x = _lrelu(x, 0.01)
        x_ref[...] = x
        h = jnp.dot(x, W_ref[...], preferred_element_type=jnp.float32)
        h_ref[...] = h
        a_ref[...] = jnp.dot(h, att_ref[...], preferred_element_type=jnp.float32)

    return pl.pallas_call(
        body,
        grid=(_NP // _BC,),
        in_specs=[
            pl.BlockSpec((1, _BC, 128), lambda j: (j // 16, j % 16, 0)),
            pl.BlockSpec((1, _BC, 1), lambda j: (j // 16, j, 0)),
            pl.BlockSpec((_BC, 128), lambda j: (j, 0)),
            pl.BlockSpec((1, 128), lambda j: (0, 0)),
            pl.BlockSpec((128, F), lambda j: (0, 0)),
            pl.BlockSpec((F, 2), lambda j: (0, 0)),
        ],
        out_specs=[
            pl.BlockSpec((_BC, 128), lambda j: (j, 0)),
            pl.BlockSpec((_BC, F), lambda j: (j, 0)),
            pl.BlockSpec((_BC, 2), lambda j: (j, 0)),
        ],
        out_shape=[
            jax.ShapeDtypeStruct((_NP, 128), jnp.float32),
            jax.ShapeDtypeStruct((_NP, F), jnp.float32),
            jax.ShapeDtypeStruct((_NP, 2), jnp.float32),
        ],
    )(parts, s2, res, b.reshape(1, -1), W, att2)


def _combine(parts, s2, res, b):
    def body(p_ref, s_ref, res_ref, b_ref, x_ref):
        num = p_ref[0]
        s = s_ref[0]
        x_ref[...] = num / (s + 1e-16) + res_ref[...] + b_ref[...]

    return pl.pallas_call(
        body,
        grid=(_NP // _BC,),
        in_specs=[
            pl.BlockSpec((1, _BC, 128), lambda j: (j // 16, j % 16, 0)),
            pl.BlockSpec((1, _BC, 1), lambda j: (j // 16, j, 0)),
            pl.BlockSpec((_BC, 128), lambda j: (j, 0)),
            pl.BlockSpec((1, 128), lambda j: (0, 0)),
        ],
        out_specs=pl.BlockSpec((_BC, 128), lambda j: (j, 0)),
        out_shape=jax.ShapeDtypeStruct((_NP, 128), jnp.float32),
    )(parts, s2, res, b.reshape(1, -1))


def _pad_edges(src, dst, e_pad):
    pad = e_pad - src.shape[0]
    src_p = jnp.concatenate([src, jnp.zeros((pad,), src.dtype)])
    dst_p = jnp.concatenate([dst, jnp.full((pad,), _N, dst.dtype)])
    return src_p, dst_p


def _pad_rows(x):
    return jnp.concatenate(
        [x, jnp.zeros((_NP - x.shape[0], x.shape[1]), x.dtype)])


def kernel(x_s, edge_index_s, edge_attr_s, x_s_batch, x_t, edge_index_t,
           x_t_batch, prot_esm, interaction_id, params):
    del interaction_id
    p = params
    ids_s = x_s_batch.reshape(-1, 1).astype(jnp.int32)
    ids_t = x_t_batch.reshape(-1, 1).astype(jnp.int32)

    cpt_s = 80
    cpt_t = 158
    src_s, dst_s = _pad_edges(edge_index_s[0], edge_index_s[1], 16 * cpt_s * _K)
    src_t, dst_t = _pad_edges(edge_index_t[0], edge_index_t[1], 16 * cpt_t * _K)
    zeros = jnp.zeros((_NP, 128), jnp.float32)

    Ke = jnp.stack([p[f"s{i}"]["W_e"] @ p[f"s{i}"]["att_e"] for i in range(4)],
                   axis=1)
    ae_all = _linear(edge_attr_s, Ke, bn=1000)
    ae_pad = jnp.zeros((16 * cpt_s * _K, 4), jnp.float32)
    ae_pad = ae_pad.at[:ae_all.shape[0]].set(ae_all)

    x = _pad_rows(x_s)
    pr = p["s0"]
    att2 = jnp.stack([pr["att_src"], pr["att_dst"]], axis=1)
    h, a2 = _pre_call(x, pr["W"], att2)
    for i in range(4):
        pr = p[f"s{i}"]
        parts, s2 = _msg_parts(h, a2, ae_pad[:, i], src_s, dst_s,
                               zeros, cpt_s, True)
        if i < 3:
            nxt = p[f"s{i + 1}"]
            att2 = jnp.stack([nxt["att_src"], nxt["att_dst"]], axis=1)
            x, h, a2 = _layer_step(parts, s2, x, pr["b"], nxt["W"], att2,
                                   "lrelu")
        else:
            h = _combine(parts, s2, x, pr["b"])

    mx_s, sm_s, ct_s = _pool_smiles(h[:_N], ids_s)
    x_smile, U1_xs = _smile_out(mx_s, sm_s, ct_s, p["s_out"]["W"],
                                p["s_out"]["b"], p["U1"]["W"], p["U1"]["b"])

    W1a = p["W1"]["W"][:150]
    W1b = p["W1"]["W"][150:]
    ee, ssum = _attention(x_t, ids_t, U1_xs, p["U2"]["W"],
                          p["U2"]["b"].reshape(1, -1), W1a, W1b,
                          p["W1"]["b"].reshape(1, -1), p["W2"]["W"],
                          p["W2"]["b"].reshape(1, -1))
    alpha = _alpha_norm(ee, ids_t, ssum)

    xt_pad = _pad_rows(x_t)
    pr = p["p0"]
    att2 = jnp.stack([pr["att_src"], pr["att_dst"]], axis=1)
    hp, a2 = _pre_call(xt_pad, pr["W"], att2)
    xp = _linear(xt_pad, pr["res_W"], bn=1280)
    for i in range(4):
        pr = p[f"p{i}"]
        parts, s2 = _msg_parts(hp, a2, src_t, src_t, dst_t,
                               zeros, cpt_t, False)
        if i < 3:
            nxt = p[f"p{i + 1}"]
            att2 = jnp.stack([nxt["att_src"], nxt["att_dst"]], axis=1)
            xp, hp, a2 = _layer_step(parts, s2, xp, pr["b"], nxt["W"], att2,
                                     "lrelu")
        else:
            hp = _combine(parts, s2, xp, pr["b"])

    mx_p = _pool_prot(hp[:_N], alpha, ids_t)

    out = _head(mx_p, p["p_out"]["W"], p["p_out"]["b"], x_smile, prot_esm,
                p["l1"]["W"], p["l1"]["b"], p["l2"]["W"], p["l2"]["b"],
                p["l3"]["W"], p["l3"]["b"], p["out"]["W"], p["out"]["b"])
    return out, alpha
